# baseline (speedup 1.0000x reference)
import jax
import jax.numpy as jnp
from jax.experimental import pallas as pl
from jax.experimental.pallas import tpu as pltpu

BETA_ = 0.1
B_, N_, T_, H_, D_, S_ = 4, 4, 2048, 1024, 8, 32
NSLOT_ = 2


def _log_sigmoid(x):
    return jnp.minimum(x, 0.0) - jnp.log1p(jnp.exp(-jnp.abs(x)))


def _tpo_kernel(hid_hbm, pol_ref, ref_ref, step_ref, labels_ref,
                loss_ref, chosen_ref, rejected_ref,
                buf, sem, hid_acc, cnt_acc):
    B, N, T, H, D, S = B_, N_, T_, H_, D_, S_
    BN = B * N

    def copy(row, slot):
        return pltpu.make_async_copy(hid_hbm.at[row], buf.at[slot], sem.at[slot])

    s_iota = jax.lax.broadcasted_iota(jnp.int32, (S, T), 0)

    copy(0, 0).start()
    for row in range(BN):
        slot = row % NSLOT_
        if row + 1 < BN:
            copy(row + 1, (row + 1) % NSLOT_).start()
        step_row = step_ref[row, 0, :]
        onehot = (s_iota == step_row[None, :]).astype(jnp.float32)
        copy(row, slot).wait()
        hid_acc[row] = jnp.dot(onehot.astype(jnp.bfloat16),
                               buf[slot].astype(jnp.bfloat16),
                               preferred_element_type=jnp.float32)
        cnt_acc[row] = jnp.sum(onehot, axis=1)

    log_sum_l = []
    for j in range(BN):
        st = step_ref[j, 0, :]
        oh = (s_iota == st[None, :]).astype(jnp.float32)
        lg = pol_ref[j] - ref_ref[j]
        log_sum_l.append(jnp.dot(oh, lg, preferred_element_type=jnp.float32))
    log_sum = jnp.stack(log_sum_l).reshape(B, N, S, D)

    hid_sum = hid_acc[...].reshape(B, N, S, H)
    cnt = cnt_acc[...].reshape(B, N, S)
    labels = labels_ref[...]

    safe_cnt = jnp.maximum(cnt, 1.0)
    hid_mean = hid_sum / safe_cnt[..., None]
    ref_mean = hid_mean[:, 0]
    ref_cnt = cnt[:, 0]

    dot = jnp.sum(hid_mean * ref_mean[:, None, :, :], axis=-1)
    nx = jnp.sqrt(jnp.sum(hid_mean * hid_mean, axis=-1))
    ny = nx[:, 0]
    cos = dot / jnp.maximum(nx * ny[:, None, :], 1e-8)

    steps = jax.lax.broadcasted_iota(jnp.int32, (B, N, S), 2)
    valid_w = (cnt > 0) & (ref_cnt[:, None, :] > 0) & (steps >= 1)
    w = jnp.where(valid_w, cos + 1.0, 0.0)

    total_w = jnp.sum(w, axis=-1)
    log_mean = log_sum / safe_cnt[..., None]
    weighted = jnp.sum(w[..., None] * log_mean, axis=2)
    denom = jnp.where(total_w > 0, total_w, 1.0)
    weighted_logits = jnp.where(total_w[..., None] > 0,
                                weighted / denom[..., None], 0.0)
    text_logits = jnp.mean(weighted_logits, axis=-1)

    diff = text_logits[:, :, None] - text_logits[:, None, :]
    ld = labels[:, :, None] - labels[:, None, :]
    pl_loss = -_log_sigmoid(diff * jnp.sign(ld))
    lrl = jnp.mean(jnp.sum(pl_loss, axis=(1, 2)) / (N * (N - 1)))
    loss = -_log_sigmoid(BETA_ * lrl)

    chosen = jnp.sum(log_sum[:, 0]) / (B * T * D)
    rejected = jnp.sum(log_sum[:, N - 1]) / (B * T * D)

    loss_ref[...] = jnp.reshape(loss, (1, 1))
    chosen_ref[...] = jnp.reshape(chosen, (1, 1))
    rejected_ref[...] = jnp.reshape(rejected, (1, 1))


def kernel(policy_responses_logps, reference_responses_logps, hidden_state,
           step_index, labels):
    B, N, T, H = hidden_state.shape
    D = policy_responses_logps.shape[-1]
    S = S_
    BN = B * N

    hid = hidden_state.reshape(BN, T, H)
    pol = policy_responses_logps.reshape(BN, T, D)
    ref = reference_responses_logps.reshape(BN, T, D)
    step = step_index.reshape(BN, 1, T)

    out_shape = (
        jax.ShapeDtypeStruct((1, 1), jnp.float32),
        jax.ShapeDtypeStruct((1, 1), jnp.float32),
        jax.ShapeDtypeStruct((1, 1), jnp.float32),
    )
    loss, chosen, rejected = pl.pallas_call(
        _tpo_kernel,
        in_specs=[
            pl.BlockSpec(memory_space=pltpu.MemorySpace.HBM),
            pl.BlockSpec((BN, T, D), lambda: (0, 0, 0)),
            pl.BlockSpec((BN, T, D), lambda: (0, 0, 0)),
            pl.BlockSpec((BN, 1, T), lambda: (0, 0, 0)),
            pl.BlockSpec((B, N), lambda: (0, 0)),
        ],
        out_specs=[
            pl.BlockSpec((1, 1), lambda: (0, 0)),
            pl.BlockSpec((1, 1), lambda: (0, 0)),
            pl.BlockSpec((1, 1), lambda: (0, 0)),
        ],
        out_shape=out_shape,
        scratch_shapes=[
            pltpu.VMEM((NSLOT_, T, H), jnp.float32),
            pltpu.SemaphoreType.DMA((NSLOT_,)),
            pltpu.VMEM((BN, S, H), jnp.float32),
            pltpu.VMEM((BN, S), jnp.float32),
        ],
    )(hid, pol, ref, step, labels)
    return loss[0, 0], chosen[0, 0], rejected[0, 0]
